# baseline (speedup 1.0000x reference)
import functools

import jax
import jax.numpy as jnp
from jax import lax
from jax.experimental import pallas as pl
from jax.experimental.pallas import tpu as pltpu
from jax.experimental.pallas import tpu_sc as plsc

_INFO = plsc.get_sparse_core_info()
_NC, _NS, _L = _INFO.num_cores, _INFO.num_subcores, _INFO.num_lanes
_NW = _NC * _NS
_NBUF = 2


def _make_lookup(n_rows: int, n_cols: int, n_values: int, chunk_rows: int):
    cols_per_w = n_cols // _NW
    assert cols_per_w * _NW == n_cols and cols_per_w % 128 == 0
    n_chunks = n_rows // chunk_rows
    assert n_chunks * chunk_rows == n_rows and n_chunks >= _NBUF
    assert chunk_rows % 8 == 0
    kpr = cols_per_w // _L
    assert kpr & (kpr - 1) == 0
    kb = kpr.bit_length() - 1
    mesh = plsc.VectorSubcoreMesh(core_axis_name="c", subcore_axis_name="s")
    n_pad = 16 * ((n_values + 15) // 16)

    @functools.partial(
        pl.kernel,
        mesh=mesh,
        out_type=jax.ShapeDtypeStruct((n_rows, n_cols), jnp.float32),
        scratch_types=[
            pltpu.VMEM((n_pad,), jnp.float32),
            pltpu.VMEM((_NBUF, chunk_rows, cols_per_w), jnp.int32),
            pltpu.VMEM((_NBUF, chunk_rows, cols_per_w), jnp.float32),
            pltpu.SemaphoreType.DMA((_NBUF,)),
            pltpu.SemaphoreType.DMA((_NBUF,)),
        ],
        compiler_params=pltpu.CompilerParams(needs_layout_passes=False),
    )
    def lookup(values_hbm, idx_hbm, out_hbm, table_v, idx_v, out_v, isem, osem):
        wid = lax.axis_index("s") * _NC + lax.axis_index("c")
        cols = pl.ds(wid * cols_per_w, cols_per_w)

        for b in range(_NBUF):
            pltpu.async_copy(
                idx_hbm.at[pl.ds(b * chunk_rows, chunk_rows), cols],
                idx_v.at[b],
                isem.at[b],
            )
        pltpu.sync_copy(values_hbm, table_v.at[pl.ds(0, n_values)])

        @pl.loop(0, n_chunks)
        def _chunk(g):
            b = g & (_NBUF - 1)
            rows = pl.ds(g * chunk_rows, chunk_rows)
            pltpu.make_async_copy(
                idx_hbm.at[rows, cols], idx_v.at[b], isem.at[b]
            ).wait()

            @pl.when(g >= _NBUF)
            def _():
                pltpu.make_async_copy(
                    out_v.at[b], out_hbm.at[rows, cols], osem.at[b]
                ).wait()

            @plsc.parallel_loop(0, chunk_rows * kpr, unroll=8)
            def _vec(i):
                r = i >> kb
                c = (i & (kpr - 1)) * _L
                iv = idx_v[b, r, pl.ds(c, _L)]
                out_v[b, r, pl.ds(c, _L)] = plsc.load_gather(table_v, [iv])

            pltpu.async_copy(out_v.at[b], out_hbm.at[rows, cols], osem.at[b])

            @pl.when(g + _NBUF < n_chunks)
            def _():
                nxt = pl.ds((g + _NBUF) * chunk_rows, chunk_rows)
                pltpu.async_copy(idx_hbm.at[nxt, cols], idx_v.at[b], isem.at[b])

        for g in range(n_chunks - _NBUF, n_chunks):
            rows = pl.ds(g * chunk_rows, chunk_rows)
            pltpu.make_async_copy(
                out_v.at[g & (_NBUF - 1)], out_hbm.at[rows, cols],
                osem.at[g & (_NBUF - 1)],
            ).wait()

    return lookup


def kernel(values, index):
    n_rows, n_cols = index.shape
    idx_t = index.T
    lookup = _make_lookup(n_cols, n_rows, values.shape[0], chunk_rows=8)
    out_t = lookup(values, idx_t)
    return out_t.T


# speedup vs baseline: 1.1688x; 1.1688x over previous
import functools

import jax
import jax.numpy as jnp
from jax import lax
from jax.experimental import pallas as pl
from jax.experimental.pallas import tpu as pltpu
from jax.experimental.pallas import tpu_sc as plsc

_INFO = plsc.get_sparse_core_info()
_NC, _NS, _L = _INFO.num_cores, _INFO.num_subcores, _INFO.num_lanes
_NW = _NC * _NS
_NBUF = 2


def _make_lookup(n_rows: int, n_cols: int, n_values: int, chunk_rows: int):
    cols_per_w = n_cols // _NW
    assert cols_per_w * _NW == n_cols and cols_per_w % 128 == 0
    n_chunks = n_rows // chunk_rows
    assert n_chunks * chunk_rows == n_rows and n_chunks >= _NBUF
    assert chunk_rows % 8 == 0
    kpr = cols_per_w // _L
    assert kpr & (kpr - 1) == 0
    kb = kpr.bit_length() - 1
    mesh = plsc.VectorSubcoreMesh(core_axis_name="c", subcore_axis_name="s")
    n_pad = 16 * ((n_values + 15) // 16)

    @functools.partial(
        pl.kernel,
        mesh=mesh,
        out_type=jax.ShapeDtypeStruct((n_rows, n_cols), jnp.float32),
        scratch_types=[
            pltpu.VMEM((n_pad,), jnp.float32),
            pltpu.VMEM((_NBUF, chunk_rows, cols_per_w), jnp.int32),
            pltpu.VMEM((_NBUF, chunk_rows, cols_per_w), jnp.float32),
            pltpu.SemaphoreType.DMA((_NBUF,)),
            pltpu.SemaphoreType.DMA((_NBUF,)),
        ],
        compiler_params=pltpu.CompilerParams(needs_layout_passes=False),
    )
    def lookup(values_hbm, idx_hbm, out_hbm, table_v, idx_v, out_v, isem, osem):
        wid = lax.axis_index("s") * _NC + lax.axis_index("c")
        cols = pl.ds(wid * cols_per_w, cols_per_w)

        for b in range(_NBUF):
            pltpu.async_copy(
                idx_hbm.at[pl.ds(b * chunk_rows, chunk_rows), cols],
                idx_v.at[b],
                isem.at[b],
            )
        pltpu.sync_copy(values_hbm, table_v.at[pl.ds(0, n_values)])

        @pl.loop(0, n_chunks)
        def _chunk(g):
            b = g & (_NBUF - 1)
            rows = pl.ds(g * chunk_rows, chunk_rows)
            pltpu.make_async_copy(
                idx_hbm.at[rows, cols], idx_v.at[b], isem.at[b]
            ).wait()

            @pl.when(g >= _NBUF)
            def _():
                pltpu.make_async_copy(
                    out_v.at[b], out_hbm.at[rows, cols], osem.at[b]
                ).wait()

            @plsc.parallel_loop(0, chunk_rows * kpr, unroll=8)
            def _vec(i):
                r = i >> kb
                c = (i & (kpr - 1)) * _L
                iv = idx_v[b, r, pl.ds(c, _L)]
                out_v[b, r, pl.ds(c, _L)] = plsc.load_gather(table_v, [iv])

            pltpu.async_copy(out_v.at[b], out_hbm.at[rows, cols], osem.at[b])

            @pl.when(g + _NBUF < n_chunks)
            def _():
                nxt = pl.ds((g + _NBUF) * chunk_rows, chunk_rows)
                pltpu.async_copy(idx_hbm.at[nxt, cols], idx_v.at[b], isem.at[b])

        for g in range(n_chunks - _NBUF, n_chunks):
            rows = pl.ds(g * chunk_rows, chunk_rows)
            pltpu.make_async_copy(
                out_v.at[g & (_NBUF - 1)], out_hbm.at[rows, cols],
                osem.at[g & (_NBUF - 1)],
            ).wait()

    return lookup


def kernel(values, index):
    n_rows, n_cols = index.shape
    idx_t = index.T
    lookup = _make_lookup(n_cols, n_rows, values.shape[0], chunk_rows=40)
    out_t = lookup(values, idx_t)
    return out_t.T


# speedup vs baseline: 1.1746x; 1.0050x over previous
import functools

import jax
import jax.numpy as jnp
from jax import lax
from jax.experimental import pallas as pl
from jax.experimental.pallas import tpu as pltpu
from jax.experimental.pallas import tpu_sc as plsc

_INFO = plsc.get_sparse_core_info()
_NC, _NS, _L = _INFO.num_cores, _INFO.num_subcores, _INFO.num_lanes
_NW = _NC * _NS
_NBUF = 2


def _make_lookup(n_rows: int, n_cols: int, n_values: int, chunk_rows: int):
    cols_per_w = n_cols // _NW
    assert cols_per_w * _NW == n_cols and cols_per_w % 128 == 0
    n_chunks = n_rows // chunk_rows
    assert n_chunks * chunk_rows == n_rows and n_chunks >= _NBUF
    assert chunk_rows % 8 == 0
    kpr = cols_per_w // _L
    assert kpr & (kpr - 1) == 0
    kb = kpr.bit_length() - 1
    mesh = plsc.VectorSubcoreMesh(core_axis_name="c", subcore_axis_name="s")
    n_pad = 16 * ((n_values + 15) // 16)

    @functools.partial(
        pl.kernel,
        mesh=mesh,
        out_type=jax.ShapeDtypeStruct((n_rows, n_cols), jnp.float32),
        scratch_types=[
            pltpu.VMEM((n_pad,), jnp.float32),
            pltpu.VMEM((_NBUF, chunk_rows, cols_per_w), jnp.int32),
            pltpu.VMEM((_NBUF, chunk_rows, cols_per_w), jnp.float32),
            pltpu.SemaphoreType.DMA((_NBUF,)),
            pltpu.SemaphoreType.DMA((_NBUF,)),
        ],
        compiler_params=pltpu.CompilerParams(needs_layout_passes=False),
    )
    def lookup(values_hbm, idx_hbm, out_hbm, table_v, idx_v, out_v, isem, osem):
        wid = lax.axis_index("s") * _NC + lax.axis_index("c")
        cols = pl.ds(wid * cols_per_w, cols_per_w)

        for b in range(_NBUF):
            pltpu.async_copy(
                idx_hbm.at[pl.ds(b * chunk_rows, chunk_rows), cols],
                idx_v.at[b],
                isem.at[b],
            )
        pltpu.sync_copy(values_hbm, table_v.at[pl.ds(0, n_values)])

        @pl.loop(0, n_chunks)
        def _chunk(g):
            b = g & (_NBUF - 1)
            rows = pl.ds(g * chunk_rows, chunk_rows)
            pltpu.make_async_copy(
                idx_hbm.at[rows, cols], idx_v.at[b], isem.at[b]
            ).wait()

            @pl.when(g >= _NBUF)
            def _():
                pltpu.make_async_copy(
                    out_v.at[b], out_hbm.at[rows, cols], osem.at[b]
                ).wait()

            @plsc.parallel_loop(0, chunk_rows * kpr, unroll=16)
            def _vec(i):
                r = i >> kb
                c = (i & (kpr - 1)) * _L
                iv = idx_v[b, r, pl.ds(c, _L)]
                out_v[b, r, pl.ds(c, _L)] = plsc.load_gather(table_v, [iv])

            pltpu.async_copy(out_v.at[b], out_hbm.at[rows, cols], osem.at[b])

            @pl.when(g + _NBUF < n_chunks)
            def _():
                nxt = pl.ds((g + _NBUF) * chunk_rows, chunk_rows)
                pltpu.async_copy(idx_hbm.at[nxt, cols], idx_v.at[b], isem.at[b])

        for g in range(n_chunks - _NBUF, n_chunks):
            rows = pl.ds(g * chunk_rows, chunk_rows)
            pltpu.make_async_copy(
                out_v.at[g & (_NBUF - 1)], out_hbm.at[rows, cols],
                osem.at[g & (_NBUF - 1)],
            ).wait()

    return lookup


def kernel(values, index):
    n_rows, n_cols = index.shape
    idx_t = index.T
    lookup = _make_lookup(n_cols, n_rows, values.shape[0], chunk_rows=40)
    out_t = lookup(values, idx_t)
    return out_t.T
